# Initial kernel scaffold; baseline (speedup 1.0000x reference)
import jax
import jax.numpy as jnp
from jax.experimental import pallas as pl


def kernel(inputs):
    raise NotImplementedError("write your pallas kernel here")


# speedup vs baseline: 91.1866x; 91.1866x over previous
import functools

import jax
import jax.numpy as jnp
from jax import lax
from jax.experimental import pallas as pl
from jax.experimental.pallas import tpu as pltpu
from jax.experimental.pallas import tpu_sc as plsc

NC, NS, L = 2, 16, 16
NW = NC * NS
B, S, C = 4, 8192, 768
K = 8
G = C // L
ROWS = B * S
ROWS_TC = 2 * S
RPW = (ROWS - ROWS_TC) // NW
SLAB = 64
NSLAB = RPW // SLAB
MPW = G // NS
TCB = 512
TC_STEPS = S // TCB

_NEG = float("-inf")


_SORT8 = [(0, 1), (2, 3), (0, 2), (1, 3), (1, 2),
          (4, 5), (6, 7), (4, 6), (5, 7), (5, 6),
          (0, 4), (1, 5), (2, 6), (3, 7),
          (2, 4), (3, 5),
          (1, 2), (3, 4), (5, 6)]
_BSORT8 = [(0, 4), (1, 5), (2, 6), (3, 7),
           (0, 2), (1, 3), (4, 6), (5, 7),
           (0, 1), (2, 3), (4, 5), (6, 7)]


def _merge_sorted(m, s):
    t = [jnp.maximum(m[i], s[7 - i]) for i in range(K)]
    for i, j in _BSORT8:
        hi = jnp.maximum(t[i], t[j])
        lo = jnp.minimum(t[i], t[j])
        t[i], t[j] = hi, lo
    return t


def _sort8(xs):
    s = list(xs)
    for i, j in _SORT8:
        hi = jnp.maximum(s[i], s[j])
        lo = jnp.minimum(s[i], s[j])
        s[i], s[j] = hi, lo
    return s


def _sort_chunk_merge(m, xs):
    return _merge_sorted(m, _sort8(xs))


def _fused_body(x_hbm, out_hbm, buf0, buf1, acc, shared, mbuf, outbuf,
                sem0, sem1):
    cid = lax.axis_index("c")
    sid = lax.axis_index("s")
    wid = cid * NS + sid
    row0 = ROWS_TC + wid * RPW

    neg = jnp.full((L,), _NEG, jnp.float32)

    def init_g(g, carry):
        for k in range(K):
            acc[pl.ds(g * K * L + k * L, L)] = neg
        return carry

    lax.fori_loop(0, G, init_g, 0)

    def src(j):
        return x_hbm.at[pl.ds(row0 + j * SLAB, SLAB), :]

    def compute(buf):
        def group_loop(g, carry):
            m0 = tuple(acc[pl.ds(g * K * L + k * L, L)] for k in range(K))

            def chunk(cidx, m):
                xs = [buf[cidx * 8 + r, pl.ds(g * L, L)] for r in range(8)]
                return tuple(_sort_chunk_merge(list(m), xs))

            m = lax.fori_loop(0, SLAB // 8, chunk, m0)
            for k in range(K):
                acc[pl.ds(g * K * L + k * L, L)] = m[k]
            return carry

        lax.fori_loop(0, G, group_loop, 0)

    pltpu.async_copy(src(0), buf0, sem0)

    def slab_pair(p, carry):
        j0 = 2 * p
        pltpu.make_async_copy(src(j0), buf0, sem0).wait()
        pltpu.async_copy(src(j0 + 1), buf1, sem1)
        compute(buf0)
        pltpu.make_async_copy(src(j0 + 1), buf1, sem1).wait()

        @pl.when(p < NSLAB // 2 - 1)
        def _():
            pltpu.async_copy(src(j0 + 2), buf0, sem0)

        compute(buf1)
        return carry

    lax.fori_loop(0, NSLAB // 2, slab_pair, 0)

    pltpu.sync_copy(acc, shared.at[sid])
    plsc.subcore_barrier()

    i16 = lax.iota(jnp.int32, L)

    def task(i, carry):
        g = sid * MPW + i
        pltpu.sync_copy(shared.at[:, pl.ds(g * K * L, K * L)], mbuf)
        m = [mbuf[0, pl.ds(k * L, L)] for k in range(K)]
        for q in range(1, NS):
            m = _merge_sorted(m, [mbuf[q, pl.ds(k * L, L)] for k in range(K)])
        for j in range(K):
            a = jnp.full((L,), m[0][2 * j])
            bvec = jnp.full((L,), m[0][2 * j + 1])
            for k in range(1, K):
                a = jnp.where((i16 % K) == k, jnp.full((L,), m[k][2 * j]), a)
                bvec = jnp.where((i16 % K) == k,
                                 jnp.full((L,), m[k][2 * j + 1]), bvec)
            outbuf[pl.ds(j * L, L)] = jnp.where(i16 < K, a, bvec)
        pltpu.sync_copy(outbuf, out_hbm.at[cid, pl.ds(g * K * L, K * L)])
        return carry

    lax.fori_loop(0, MPW, task, 0)


def _tc_body(x_ref, out_ref, acc_ref):
    s = pl.program_id(1)

    for lg in range(6):
        sl = pl.ds(lg * 128, 128)
        neg = jnp.full((8, 128), _NEG, jnp.float32)
        m = tuple(jnp.where(s == 0, neg, acc_ref[k, :, sl]) for k in range(K))

        def chunk(ci, m):
            xs = [x_ref[pl.ds(ci * 64 + r * 8, 8), sl] for r in range(8)]
            return tuple(_sort_chunk_merge(list(m), xs))

        m = lax.fori_loop(0, TCB // 64, chunk, m)
        for k in range(K):
            acc_ref[k, :, sl] = m[k]

    @pl.when(s == TC_STEPS - 1)
    def _():
        m = [acc_ref[k] for k in range(K)]
        for d in (4, 2, 1):
            rolled = [pltpu.roll(mk, d, axis=0) for mk in m]
            m = _merge_sorted(m, rolled)
        a = jnp.concatenate([mk[0:1, :] for mk in m], axis=0)
        out_ref[0] = jnp.transpose(a)


@functools.cache
def _build_sc():
    mesh = plsc.VectorSubcoreMesh(core_axis_name="c", subcore_axis_name="s",
                                  num_cores=NC, num_subcores=NS)
    return pl.kernel(
        _fused_body,
        out_type=jax.ShapeDtypeStruct((2, C * K), jnp.float32),
        mesh=mesh,
        scratch_types=[
            pltpu.VMEM((SLAB, C), jnp.float32),
            pltpu.VMEM((SLAB, C), jnp.float32),
            pltpu.VMEM((G * K * L,), jnp.float32),
            pltpu.VMEM_SHARED((NS, G * K * L), jnp.float32),
            pltpu.VMEM((NS, K * L), jnp.float32),
            pltpu.VMEM((K * L,), jnp.float32),
            pltpu.SemaphoreType.DMA,
            pltpu.SemaphoreType.DMA,
        ],
    )


@functools.cache
def _build_tc():
    return pl.pallas_call(
        _tc_body,
        grid=(2, TC_STEPS),
        in_specs=[pl.BlockSpec((TCB, C), lambda b, s: (b * TC_STEPS + s, 0))],
        out_specs=pl.BlockSpec((1, C, K), lambda b, s: (b, 0, 0)),
        out_shape=jax.ShapeDtypeStruct((2, C, K), jnp.float32),
        scratch_shapes=[pltpu.VMEM((K, 8, C), jnp.float32)],
    )


@jax.jit
def kernel(inputs):
    x2 = inputs.reshape(ROWS, C)
    sc_out = _build_sc()(x2)
    tc_out = _build_tc()(x2)
    return jnp.concatenate([tc_out.reshape(2, C * K), sc_out], axis=0)
